# baseline (speedup 1.0000x reference)
import functools

import jax
import jax.numpy as jnp
from jax import lax
from jax.experimental import pallas as pl
from jax.experimental.pallas import tpu as pltpu
from jax.experimental.pallas import tpu_sc as plsc

_N = 1024
_K = 64
_TB = 1024
_NCHUNK = 2

_info = plsc.get_sparse_core_info()
_NC, _NS = _info.num_cores, _info.num_subcores
_NW = _NC * _NS


def _argmin_body(x, cb, xsq):
    sim2 = jnp.dot(x + x, cb, preferred_element_type=jnp.float32)
    csq = jnp.sum(cb * cb, axis=0, keepdims=True)
    dist = (xsq.reshape(_TB, 1) + csq) - sim2
    m = jnp.min(dist, axis=1, keepdims=True)
    ids = jax.lax.broadcasted_iota(jnp.int32, (_TB, _N), 1)
    return jnp.min(jnp.where(dist == m, ids, _N), axis=1)


def _argmin_block_cbt(x_ref, xsq_ref, cb_ref, idx_ref, cbt_ref):
    i = pl.program_id(0)
    cb = cb_ref[...]
    idx_ref[...] = _argmin_body(x_ref[...], cb, xsq_ref[...])

    @pl.when(i == 0)
    def _():
        cbt_ref[...] = cb.T


def _argmin_block(x_ref, xsq_ref, cb_ref, idx_ref):
    idx_ref[...] = _argmin_body(x_ref[...], cb_ref[...], xsq_ref[...])


def _tc_argmin(flat, xsq, codebook, with_cbt):
    t = flat.shape[0]
    grid = t // _TB
    in_specs = [
        pl.BlockSpec((_TB, _K), lambda i: (i, 0)),
        pl.BlockSpec((_TB,), lambda i: (i,)),
        pl.BlockSpec((_K, _N), lambda i: (0, 0)),
    ]
    if with_cbt:
        return pl.pallas_call(
            _argmin_block_cbt,
            grid=(grid,),
            in_specs=in_specs,
            out_specs=[
                pl.BlockSpec((_TB,), lambda i: (i,)),
                pl.BlockSpec((_N, _K), lambda i: (0, 0)),
            ],
            out_shape=[
                jax.ShapeDtypeStruct((t,), jnp.int32),
                jax.ShapeDtypeStruct((_N, _K), jnp.float32),
            ],
        )(flat, xsq, codebook)
    return pl.pallas_call(
        _argmin_block,
        grid=(grid,),
        in_specs=in_specs,
        out_specs=pl.BlockSpec((_TB,), lambda i: (i,)),
        out_shape=jax.ShapeDtypeStruct((t,), jnp.int32),
    )(flat, xsq, codebook)


def _sc_gather(table, idx, t):
    bpw = t // _NW
    mesh = plsc.VectorSubcoreMesh(core_axis_name="c", subcore_axis_name="s")

    @functools.partial(
        pl.kernel, mesh=mesh,
        compiler_params=pltpu.CompilerParams(use_tc_tiling_on_sc=False),
        out_type=jax.ShapeDtypeStruct((t, _K), jnp.float32),
        scratch_types=[
            pltpu.VMEM((bpw,), jnp.int32),
            pltpu.VMEM((bpw, _K), jnp.float32),
            pltpu.SemaphoreType.DMA,
        ],
    )
    def gather_kernel(table_hbm, idx_hbm, out_hbm, idx_v, rows_v, sem):
        wid = lax.axis_index("s") * _NC + lax.axis_index("c")
        base = wid * bpw
        pltpu.sync_copy(idx_hbm.at[pl.ds(base, bpw)], idx_v)
        pltpu.async_copy(table_hbm.at[idx_v], rows_v, sem).wait()
        pltpu.sync_copy(rows_v, out_hbm.at[pl.ds(base, bpw)])

    return gather_kernel(table, idx)


def kernel(z, codebook):
    shape = z.shape
    flat = z.reshape(-1, _K)
    t = flat.shape[0]
    xsq = jnp.sum(flat ** 2, axis=1)
    tc = t // _NCHUNK
    qs = []
    cbt = None
    for c in range(_NCHUNK):
        chunk = lax.slice(flat, (c * tc, 0), ((c + 1) * tc, _K))
        xchunk = lax.slice(xsq, (c * tc,), ((c + 1) * tc,))
        if c == 0:
            idx, cbt = _tc_argmin(chunk, xchunk, codebook, True)
        else:
            idx = _tc_argmin(chunk, xchunk, codebook, False)
        qs.append(_sc_gather(cbt, idx, tc))
    out = jnp.concatenate(qs, axis=0)
    return out.reshape(shape)


# speedup vs baseline: 1.1941x; 1.1941x over previous
import functools

import jax
import jax.numpy as jnp
from jax import lax
from jax.experimental import pallas as pl
from jax.experimental.pallas import tpu as pltpu
from jax.experimental.pallas import tpu_sc as plsc

_N = 1024
_K = 64
_TB = 1024
_NCHUNK = 1

_info = plsc.get_sparse_core_info()
_NC, _NS = _info.num_cores, _info.num_subcores
_NW = _NC * _NS


def _argmin_body(x, cb, xsq):
    sim2 = jnp.dot(x + x, cb, preferred_element_type=jnp.float32)
    csq = jnp.sum(cb * cb, axis=0, keepdims=True)
    dist = (xsq.reshape(_TB, 1) + csq) - sim2
    m = jnp.min(dist, axis=1, keepdims=True)
    ids = jax.lax.broadcasted_iota(jnp.int32, (_TB, _N), 1)
    return jnp.min(jnp.where(dist == m, ids, _N), axis=1)


def _argmin_block_cbt(x_ref, xsq_ref, cb_ref, idx_ref, cbt_ref):
    i = pl.program_id(0)
    cb = cb_ref[...]
    idx_ref[...] = _argmin_body(x_ref[...], cb, xsq_ref[...])

    @pl.when(i == 0)
    def _():
        cbt_ref[...] = cb.T


def _argmin_block(x_ref, xsq_ref, cb_ref, idx_ref):
    idx_ref[...] = _argmin_body(x_ref[...], cb_ref[...], xsq_ref[...])


def _tc_argmin(flat, xsq, codebook, with_cbt):
    t = flat.shape[0]
    grid = t // _TB
    in_specs = [
        pl.BlockSpec((_TB, _K), lambda i: (i, 0)),
        pl.BlockSpec((_TB,), lambda i: (i,)),
        pl.BlockSpec((_K, _N), lambda i: (0, 0)),
    ]
    if with_cbt:
        return pl.pallas_call(
            _argmin_block_cbt,
            grid=(grid,),
            in_specs=in_specs,
            out_specs=[
                pl.BlockSpec((_TB,), lambda i: (i,)),
                pl.BlockSpec((_N, _K), lambda i: (0, 0)),
            ],
            out_shape=[
                jax.ShapeDtypeStruct((t,), jnp.int32),
                jax.ShapeDtypeStruct((_N, _K), jnp.float32),
            ],
        )(flat, xsq, codebook)
    return pl.pallas_call(
        _argmin_block,
        grid=(grid,),
        in_specs=in_specs,
        out_specs=pl.BlockSpec((_TB,), lambda i: (i,)),
        out_shape=jax.ShapeDtypeStruct((t,), jnp.int32),
    )(flat, xsq, codebook)


def _sc_gather(table, idx, t):
    bpw = t // _NW
    mesh = plsc.VectorSubcoreMesh(core_axis_name="c", subcore_axis_name="s")

    @functools.partial(
        pl.kernel, mesh=mesh,
        compiler_params=pltpu.CompilerParams(use_tc_tiling_on_sc=False),
        out_type=jax.ShapeDtypeStruct((t, _K), jnp.float32),
        scratch_types=[
            pltpu.VMEM((bpw,), jnp.int32),
            pltpu.VMEM((bpw, _K), jnp.float32),
            pltpu.SemaphoreType.DMA,
        ],
    )
    def gather_kernel(table_hbm, idx_hbm, out_hbm, idx_v, rows_v, sem):
        wid = lax.axis_index("s") * _NC + lax.axis_index("c")
        base = wid * bpw
        pltpu.sync_copy(idx_hbm.at[pl.ds(base, bpw)], idx_v)
        pltpu.async_copy(table_hbm.at[idx_v], rows_v, sem).wait()
        pltpu.sync_copy(rows_v, out_hbm.at[pl.ds(base, bpw)])

    return gather_kernel(table, idx)


def kernel(z, codebook):
    shape = z.shape
    flat = z.reshape(-1, _K)
    t = flat.shape[0]
    xsq = jnp.sum(flat ** 2, axis=1)
    tc = t // _NCHUNK
    qs = []
    cbt = None
    for c in range(_NCHUNK):
        chunk = lax.slice(flat, (c * tc, 0), ((c + 1) * tc, _K))
        xchunk = lax.slice(xsq, (c * tc,), ((c + 1) * tc,))
        if c == 0:
            idx, cbt = _tc_argmin(chunk, xchunk, codebook, True)
        else:
            idx = _tc_argmin(chunk, xchunk, codebook, False)
        qs.append(_sc_gather(cbt, idx, tc))
    out = jnp.concatenate(qs, axis=0)
    return out.reshape(shape)


# speedup vs baseline: 1.2195x; 1.0212x over previous
import functools

import jax
import jax.numpy as jnp
from jax import lax
from jax.experimental import pallas as pl
from jax.experimental.pallas import tpu as pltpu
from jax.experimental.pallas import tpu_sc as plsc

_N = 1024
_K = 64
_TB = 2048
_NCHUNK = 1

_info = plsc.get_sparse_core_info()
_NC, _NS = _info.num_cores, _info.num_subcores
_NW = _NC * _NS


def _argmin_body(x, cb, xsq):
    sim2 = jnp.dot(x + x, cb, preferred_element_type=jnp.float32)
    csq = jnp.sum(cb * cb, axis=0, keepdims=True)
    dist = (xsq + csq) - sim2
    m = jnp.min(dist, axis=1, keepdims=True)
    ids = jax.lax.broadcasted_iota(jnp.int32, (1, _N), 1).astype(jnp.float32)
    idxf = jnp.min(jnp.where(dist == m, ids, float(_N)), axis=1)
    return idxf.astype(jnp.int32)


def _argmin_block_cbt(x_ref, xsq_ref, cb_ref, idx_ref, cbt_ref):
    i = pl.program_id(0)
    cb = cb_ref[...]
    idx_ref[...] = _argmin_body(x_ref[...], cb, xsq_ref[...])

    @pl.when(i == 0)
    def _():
        cbt_ref[...] = cb.T


def _argmin_block(x_ref, xsq_ref, cb_ref, idx_ref):
    idx_ref[...] = _argmin_body(x_ref[...], cb_ref[...], xsq_ref[...])


def _tc_argmin(flat, xsq, codebook, with_cbt):
    t = flat.shape[0]
    grid = t // _TB
    in_specs = [
        pl.BlockSpec((_TB, _K), lambda i: (i, 0)),
        pl.BlockSpec((_TB, 1), lambda i: (i, 0)),
        pl.BlockSpec((_K, _N), lambda i: (0, 0)),
    ]
    if with_cbt:
        return pl.pallas_call(
            _argmin_block_cbt,
            grid=(grid,),
            in_specs=in_specs,
            out_specs=[
                pl.BlockSpec((_TB,), lambda i: (i,)),
                pl.BlockSpec((_N, _K), lambda i: (0, 0)),
            ],
            out_shape=[
                jax.ShapeDtypeStruct((t,), jnp.int32),
                jax.ShapeDtypeStruct((_N, _K), jnp.float32),
            ],
        )(flat, xsq, codebook)
    return pl.pallas_call(
        _argmin_block,
        grid=(grid,),
        in_specs=in_specs,
        out_specs=pl.BlockSpec((_TB,), lambda i: (i,)),
        out_shape=jax.ShapeDtypeStruct((t,), jnp.int32),
    )(flat, xsq, codebook)


def _sc_gather(table, idx, t):
    bpw = t // _NW
    mesh = plsc.VectorSubcoreMesh(core_axis_name="c", subcore_axis_name="s")

    @functools.partial(
        pl.kernel, mesh=mesh,
        compiler_params=pltpu.CompilerParams(use_tc_tiling_on_sc=False),
        out_type=jax.ShapeDtypeStruct((t, _K), jnp.float32),
        scratch_types=[
            pltpu.VMEM((bpw,), jnp.int32),
            pltpu.VMEM((bpw, _K), jnp.float32),
            pltpu.SemaphoreType.DMA,
        ],
    )
    def gather_kernel(table_hbm, idx_hbm, out_hbm, idx_v, rows_v, sem):
        wid = lax.axis_index("s") * _NC + lax.axis_index("c")
        base = wid * bpw
        pltpu.sync_copy(idx_hbm.at[pl.ds(base, bpw)], idx_v)
        pltpu.async_copy(table_hbm.at[idx_v], rows_v, sem).wait()
        pltpu.sync_copy(rows_v, out_hbm.at[pl.ds(base, bpw)])

    return gather_kernel(table, idx)


def kernel(z, codebook):
    shape = z.shape
    flat = z.reshape(-1, _K)
    t = flat.shape[0]
    xsq = jnp.sum(flat ** 2, axis=1, keepdims=True)
    tc = t // _NCHUNK
    qs = []
    cbt = None
    for c in range(_NCHUNK):
        chunk = lax.slice(flat, (c * tc, 0), ((c + 1) * tc, _K))
        xchunk = lax.slice(xsq, (c * tc, 0), ((c + 1) * tc, 1))
        if c == 0:
            idx, cbt = _tc_argmin(chunk, xchunk, codebook, True)
        else:
            idx = _tc_argmin(chunk, xchunk, codebook, False)
        qs.append(_sc_gather(cbt, idx, tc))
    out = jnp.concatenate(qs, axis=0)
    return out.reshape(shape)
